# baseline (speedup 1.0000x reference)
import jax
import jax.numpy as jnp
from jax.experimental import pallas as pl
from jax.experimental.pallas import tpu as pltpu

_N = 10000
_E = 160000
_D = 128
_B = 4
_K = 32
_NUM_REL = 32
_LAYERS = 2
_C = 4000
_NSTEPS = _E // _C
_RB = 2000
_BD = _B * _D


def _row_get(ref, idx, col0, ncols):
    base = (idx // 8) * 8
    off = idx - base
    tile = ref[pl.ds(base, 8), col0:col0 + ncols]
    sel = jax.lax.broadcasted_iota(jnp.int32, (8, ncols), 0) == off
    return jnp.sum(jnp.where(sel, tile, 0.0), axis=0, keepdims=True)


def _row_add(ref, idx, row, col0, ncols):
    base = (idx // 8) * 8
    off = idx - base
    tile = ref[pl.ds(base, 8), col0:col0 + ncols]
    sel = jax.lax.broadcasted_iota(jnp.int32, (8, ncols), 0) == off
    upd = jnp.where(sel, jnp.broadcast_to(row, (8, ncols)), 0.0)
    ref[pl.ds(base, 8), col0:col0 + ncols] = tile + upd


def _init_body(hr_ref, qe_ref, x_ref):
    x_ref[...] = jnp.zeros_like(x_ref)
    for b in range(_B):
        hb = hr_ref[b]
        rb = hr_ref[_B + b]
        _row_add(x_ref, hb, _row_get(qe_ref, rb, 0, _D), b * _D, _D)


def _scatter_body(hr_ref, ni_ref, no_ref, et_ref, x_ref, reltab_ref, qe_ref,
                  agg_ref):
    step = pl.program_id(0)

    @pl.when(step == 0)
    def _():
        agg_ref[...] = jnp.zeros_like(agg_ref)

    def body(j, carry):
        ni = ni_ref[0, 0, j]
        no = no_ref[0, 0, j]
        t = et_ref[0, 0, j]
        xb = (ni // 8) * 8
        xo = ni - xb
        ob = (no // 8) * 8
        oo = no - ob
        xtile = x_ref[pl.ds(xb, 8), :]
        rel8 = reltab_ref[pl.ds(t * 8, 8), :]
        rolled = pltpu.roll(xtile * rel8, (oo - xo + 8) % 8, 0)
        sel = jax.lax.broadcasted_iota(jnp.int32, (8, _BD), 0) == oo
        tile = agg_ref[pl.ds(ob, 8), :]
        agg_ref[pl.ds(ob, 8), :] = tile + jnp.where(sel, rolled, 0.0)
        return carry

    jax.lax.fori_loop(0, _C, body, 0, unroll=8)

    @pl.when(step == _NSTEPS - 1)
    def _():
        for b in range(_B):
            hb = hr_ref[b]
            rb = hr_ref[_B + b]
            _row_add(agg_ref, hb, _row_get(qe_ref, rb, 0, _D), b * _D, _D)


def _dense_body(agg_ref, x_ref, w_ref, bias_ref, out_ref):
    for b in range(_B):
        cols = slice(b * _D, (b + 1) * _D)
        h = jnp.dot(agg_ref[:, cols], w_ref[...],
                    preferred_element_type=jnp.float32)
        h = jnp.maximum(h + bias_ref[...], 0.0)
        out_ref[:, cols] = x_ref[:, cols] + h


def _score_body(tr_ref, x_ref, qe_ref, w1_ref, b1_ref, w2_ref, b2_ref,
                out_ref, feat_ref):
    for j in range(_B * _K):
        b = j // _K
        tj = tr_ref[j]
        rb = tr_ref[_B * _K + b]
        feat_ref[j:j + 1, 0:_D] = _row_get(x_ref, tj, b * _D, _D)
        feat_ref[j:j + 1, _D:2 * _D] = _row_get(qe_ref, rb, 0, _D)
    h1 = jnp.dot(feat_ref[...], w1_ref[...],
                 preferred_element_type=jnp.float32)
    h1 = jnp.maximum(h1 + b1_ref[...], 0.0)
    s = jnp.sum(h1 * w2_ref[...], axis=1, keepdims=True) + b2_ref[0, 0]
    out_ref[...] = jnp.broadcast_to(s, (_B * _K, _D))


def kernel(edge_index, edge_type, h_index, t_index, r_index,
           query_emb, rel_emb, W, b, W1, b1, W2, b2):
    f32 = jnp.float32
    ni3 = edge_index[0].reshape(_NSTEPS, 1, _C)
    no3 = edge_index[1].reshape(_NSTEPS, 1, _C)
    et3 = edge_type.reshape(_NSTEPS, 1, _C)
    hr = jnp.concatenate([h_index.astype(jnp.int32),
                          r_index.astype(jnp.int32)])
    tr = jnp.concatenate([t_index.reshape(-1).astype(jnp.int32),
                          r_index.astype(jnp.int32)])

    x = pl.pallas_call(
        _init_body,
        grid_spec=pltpu.PrefetchScalarGridSpec(
            num_scalar_prefetch=1,
            grid=(1,),
            in_specs=[pl.BlockSpec((_NUM_REL, _D), lambda i, *_: (0, 0))],
            out_specs=pl.BlockSpec((_N, _BD), lambda i, *_: (0, 0)),
        ),
        out_shape=jax.ShapeDtypeStruct((_N, _BD), f32),
    )(hr, query_emb)

    smem_idx = pl.BlockSpec((1, 1, _C), lambda i, *_: (i, 0, 0),
                            memory_space=pltpu.SMEM)
    full_x = pl.BlockSpec((_N, _BD), lambda i, *_: (0, 0))

    for i in range(_LAYERS):
        reltab = jnp.repeat(jnp.tile(rel_emb[i], (1, _B)), 8, axis=0)
        agg = pl.pallas_call(
            _scatter_body,
            grid_spec=pltpu.PrefetchScalarGridSpec(
                num_scalar_prefetch=1,
                grid=(_NSTEPS,),
                in_specs=[
                    smem_idx, smem_idx, smem_idx,
                    full_x,
                    pl.BlockSpec((_NUM_REL * 8, _BD), lambda i, *_: (0, 0)),
                    pl.BlockSpec((_NUM_REL, _D), lambda i, *_: (0, 0)),
                ],
                out_specs=full_x,
            ),
            out_shape=jax.ShapeDtypeStruct((_N, _BD), f32),
        )(hr, ni3, no3, et3, x, reltab, query_emb)

        x = pl.pallas_call(
            _dense_body,
            grid=(_N // _RB,),
            in_specs=[
                pl.BlockSpec((_RB, _BD), lambda i: (i, 0)),
                pl.BlockSpec((_RB, _BD), lambda i: (i, 0)),
                pl.BlockSpec((_D, _D), lambda i: (0, 0)),
                pl.BlockSpec((1, _D), lambda i: (0, 0)),
            ],
            out_specs=pl.BlockSpec((_RB, _BD), lambda i: (i, 0)),
            out_shape=jax.ShapeDtypeStruct((_N, _BD), f32),
        )(agg, x, W[i], b[i].reshape(1, _D))

    out = pl.pallas_call(
        _score_body,
        grid_spec=pltpu.PrefetchScalarGridSpec(
            num_scalar_prefetch=1,
            grid=(1,),
            in_specs=[
                full_x,
                pl.BlockSpec((_NUM_REL, _D), lambda i, *_: (0, 0)),
                pl.BlockSpec((2 * _D, 2 * _D), lambda i, *_: (0, 0)),
                pl.BlockSpec((1, 2 * _D), lambda i, *_: (0, 0)),
                pl.BlockSpec((1, 2 * _D), lambda i, *_: (0, 0)),
                pl.BlockSpec((1, 1), lambda i, *_: (0, 0)),
            ],
            out_specs=pl.BlockSpec((_B * _K, _D), lambda i, *_: (0, 0)),
            scratch_shapes=[pltpu.VMEM((_B * _K, 2 * _D), f32)],
        ),
        out_shape=jax.ShapeDtypeStruct((_B * _K, _D), f32),
    )(tr, x, query_emb, W1, b1.reshape(1, 2 * _D), W2.reshape(1, 2 * _D),
      b2.reshape(1, 1))

    return out[:, 0].reshape(_B, _K)


# speedup vs baseline: 1.2386x; 1.2386x over previous
import jax
import jax.numpy as jnp
from jax.experimental import pallas as pl
from jax.experimental.pallas import tpu as pltpu

_N = 10000
_E = 160000
_D = 128
_B = 4
_K = 32
_NUM_REL = 32
_LAYERS = 2
_C = 4000
_NSTEPS = _E // _C
_RB = 2000
_BD = _B * _D


def _row_get(ref, idx, col0, ncols):
    base = (idx // 8) * 8
    off = idx - base
    tile = ref[pl.ds(base, 8), col0:col0 + ncols]
    sel = jax.lax.broadcasted_iota(jnp.int32, (8, ncols), 0) == off
    return jnp.sum(jnp.where(sel, tile, 0.0), axis=0, keepdims=True)


def _row_add(ref, idx, row, col0, ncols):
    base = (idx // 8) * 8
    off = idx - base
    tile = ref[pl.ds(base, 8), col0:col0 + ncols]
    sel = jax.lax.broadcasted_iota(jnp.int32, (8, ncols), 0) == off
    upd = jnp.where(sel, jnp.broadcast_to(row, (8, ncols)), 0.0)
    ref[pl.ds(base, 8), col0:col0 + ncols] = tile + upd


def _init_body(hr_ref, qe_ref, x_ref):
    x_ref[...] = jnp.zeros_like(x_ref)
    for b in range(_B):
        hb = hr_ref[b]
        rb = hr_ref[_B + b]
        _row_add(x_ref, hb, _row_get(qe_ref, rb, 0, _D), b * _D, _D)


def _scatter_body(hr_ref, ni_ref, no_ref, et_ref, x_ref, reltab_ref, qe_ref,
                  agg_ref):
    step = pl.program_id(0)

    @pl.when(step == 0)
    def _():
        agg_ref[...] = jnp.zeros_like(agg_ref)

    def body(j, carry):
        ni = ni_ref[0, 0, j]
        no = no_ref[0, 0, j]
        t = et_ref[0, 0, j]
        rel8 = reltab_ref[pl.ds(t * 8, 8), :]
        msg = _row_get(x_ref, ni, 0, _BD) * rel8[0:1, :]
        _row_add(agg_ref, no, msg, 0, _BD)
        return carry

    jax.lax.fori_loop(0, _C, body, 0, unroll=16)

    @pl.when(step == _NSTEPS - 1)
    def _():
        for b in range(_B):
            hb = hr_ref[b]
            rb = hr_ref[_B + b]
            _row_add(agg_ref, hb, _row_get(qe_ref, rb, 0, _D), b * _D, _D)


def _dense_body(agg_ref, x_ref, w_ref, bias_ref, out_ref):
    for b in range(_B):
        cols = slice(b * _D, (b + 1) * _D)
        h = jnp.dot(agg_ref[:, cols], w_ref[...],
                    preferred_element_type=jnp.float32)
        h = jnp.maximum(h + bias_ref[...], 0.0)
        out_ref[:, cols] = x_ref[:, cols] + h


def _score_body(tr_ref, x_ref, qe_ref, w1_ref, b1_ref, w2_ref, b2_ref,
                out_ref, feat_ref):
    for j in range(_B * _K):
        b = j // _K
        tj = tr_ref[j]
        rb = tr_ref[_B * _K + b]
        feat_ref[j:j + 1, 0:_D] = _row_get(x_ref, tj, b * _D, _D)
        feat_ref[j:j + 1, _D:2 * _D] = _row_get(qe_ref, rb, 0, _D)
    h1 = jnp.dot(feat_ref[...], w1_ref[...],
                 preferred_element_type=jnp.float32)
    h1 = jnp.maximum(h1 + b1_ref[...], 0.0)
    s = jnp.sum(h1 * w2_ref[...], axis=1, keepdims=True) + b2_ref[0, 0]
    out_ref[...] = jnp.broadcast_to(s, (_B * _K, _D))


def kernel(edge_index, edge_type, h_index, t_index, r_index,
           query_emb, rel_emb, W, b, W1, b1, W2, b2):
    f32 = jnp.float32
    ni3 = edge_index[0].reshape(_NSTEPS, 1, _C)
    no3 = edge_index[1].reshape(_NSTEPS, 1, _C)
    et3 = edge_type.reshape(_NSTEPS, 1, _C)
    hr = jnp.concatenate([h_index.astype(jnp.int32),
                          r_index.astype(jnp.int32)])
    tr = jnp.concatenate([t_index.reshape(-1).astype(jnp.int32),
                          r_index.astype(jnp.int32)])

    x = pl.pallas_call(
        _init_body,
        grid_spec=pltpu.PrefetchScalarGridSpec(
            num_scalar_prefetch=1,
            grid=(1,),
            in_specs=[pl.BlockSpec((_NUM_REL, _D), lambda i, *_: (0, 0))],
            out_specs=pl.BlockSpec((_N, _BD), lambda i, *_: (0, 0)),
        ),
        out_shape=jax.ShapeDtypeStruct((_N, _BD), f32),
    )(hr, query_emb)

    smem_idx = pl.BlockSpec((1, 1, _C), lambda i, *_: (i, 0, 0),
                            memory_space=pltpu.SMEM)
    full_x = pl.BlockSpec((_N, _BD), lambda i, *_: (0, 0))

    for i in range(_LAYERS):
        reltab = jnp.repeat(jnp.tile(rel_emb[i], (1, _B)), 8, axis=0)
        agg = pl.pallas_call(
            _scatter_body,
            grid_spec=pltpu.PrefetchScalarGridSpec(
                num_scalar_prefetch=1,
                grid=(_NSTEPS,),
                in_specs=[
                    smem_idx, smem_idx, smem_idx,
                    full_x,
                    pl.BlockSpec((_NUM_REL * 8, _BD), lambda i, *_: (0, 0)),
                    pl.BlockSpec((_NUM_REL, _D), lambda i, *_: (0, 0)),
                ],
                out_specs=full_x,
            ),
            out_shape=jax.ShapeDtypeStruct((_N, _BD), f32),
        )(hr, ni3, no3, et3, x, reltab, query_emb)

        x = pl.pallas_call(
            _dense_body,
            grid=(_N // _RB,),
            in_specs=[
                pl.BlockSpec((_RB, _BD), lambda i: (i, 0)),
                pl.BlockSpec((_RB, _BD), lambda i: (i, 0)),
                pl.BlockSpec((_D, _D), lambda i: (0, 0)),
                pl.BlockSpec((1, _D), lambda i: (0, 0)),
            ],
            out_specs=pl.BlockSpec((_RB, _BD), lambda i: (i, 0)),
            out_shape=jax.ShapeDtypeStruct((_N, _BD), f32),
        )(agg, x, W[i], b[i].reshape(1, _D))

    out = pl.pallas_call(
        _score_body,
        grid_spec=pltpu.PrefetchScalarGridSpec(
            num_scalar_prefetch=1,
            grid=(1,),
            in_specs=[
                full_x,
                pl.BlockSpec((_NUM_REL, _D), lambda i, *_: (0, 0)),
                pl.BlockSpec((2 * _D, 2 * _D), lambda i, *_: (0, 0)),
                pl.BlockSpec((1, 2 * _D), lambda i, *_: (0, 0)),
                pl.BlockSpec((1, 2 * _D), lambda i, *_: (0, 0)),
                pl.BlockSpec((1, 1), lambda i, *_: (0, 0)),
            ],
            out_specs=pl.BlockSpec((_B * _K, _D), lambda i, *_: (0, 0)),
            scratch_shapes=[pltpu.VMEM((_B * _K, 2 * _D), f32)],
        ),
        out_shape=jax.ShapeDtypeStruct((_B * _K, _D), f32),
    )(tr, x, query_emb, W1, b1.reshape(1, 2 * _D), W2.reshape(1, 2 * _D),
      b2.reshape(1, 1))

    return out[:, 0].reshape(_B, _K)
